# baseline (speedup 1.0000x reference)
import functools

import jax
import jax.numpy as jnp
from jax import lax
from jax.experimental import pallas as pl
from jax.experimental.pallas import tpu as pltpu
from jax.experimental.pallas import tpu_sc as plsc

BATCH = 16384
DIM = 64
NC = 2
NS = 16
NW = NC * NS
ROWS_PER_W = BATCH // NW
CHUNK = 128
NCHUNK = ROWS_PER_W // CHUNK
GROUPS = ROWS_PER_W // 16


def _sc_body(head_hbm, tail_hbm, rel_hbm, ent_hbm, relemb_hbm, out_hbm,
             hidx, tidx, ridx, hrows, trows, rrows, oscr, sem):
    c = lax.axis_index("c")
    s = lax.axis_index("s")
    wid = s * NC + c
    blk0 = wid * NCHUNK

    pltpu.sync_copy(head_hbm.at[pl.ds(blk0, NCHUNK)], hidx)
    pltpu.sync_copy(tail_hbm.at[pl.ds(blk0, NCHUNK)], tidx)
    pltpu.sync_copy(rel_hbm.at[pl.ds(blk0, NCHUNK)], ridx)

    copies = []
    for ck in range(NCHUNK):
        dst = pl.ds(ck * CHUNK, CHUNK)
        copies.append(pltpu.async_copy(ent_hbm.at[hidx.at[ck]], hrows.at[dst], sem))
        copies.append(pltpu.async_copy(ent_hbm.at[tidx.at[ck]], trows.at[dst], sem))
        copies.append(pltpu.async_copy(relemb_hbm.at[ridx.at[ck]], rrows.at[dst], sem))
    for cp in copies:
        cp.wait()

    iota16 = lax.iota(jnp.int32, 16)

    def group(g, carry):
        rowv = g * 16 + iota16
        acc = jnp.zeros((16,), jnp.float32)
        for d in range(DIM):
            dv = jnp.full((16,), d, jnp.int32)
            h = plsc.load_gather(hrows, [rowv, dv])
            t = plsc.load_gather(trows, [rowv, dv])
            r = plsc.load_gather(rrows, [rowv, dv])
            acc = acc + h * t * r
        score = 1.0 / (1.0 + jnp.exp(-acc))
        oscr[pl.ds(pl.multiple_of(g * 16, 16), 16)] = score
        return carry

    lax.fori_loop(0, GROUPS, group, 0)

    pltpu.sync_copy(oscr, out_hbm.at[pl.ds(wid * ROWS_PER_W, ROWS_PER_W)])


@functools.partial(
    pl.kernel,
    mesh=plsc.VectorSubcoreMesh(core_axis_name="c", subcore_axis_name="s"),
    out_type=jax.ShapeDtypeStruct((BATCH,), jnp.float32),
    compiler_params=pltpu.CompilerParams(
        needs_layout_passes=False, use_tc_tiling_on_sc=False
    ),
    scratch_types=[
        pltpu.VMEM((NCHUNK, CHUNK), jnp.int32),
        pltpu.VMEM((NCHUNK, CHUNK), jnp.int32),
        pltpu.VMEM((NCHUNK, CHUNK), jnp.int32),
        pltpu.VMEM((ROWS_PER_W, DIM), jnp.float32),
        pltpu.VMEM((ROWS_PER_W, DIM), jnp.float32),
        pltpu.VMEM((ROWS_PER_W, DIM), jnp.float32),
        pltpu.VMEM((ROWS_PER_W,), jnp.float32),
        pltpu.SemaphoreType.DMA,
    ],
)
def _distmult_sc(*args):
    _sc_body(*args)


def kernel(head, tail, relation, entity_embed, relation_embed):
    head2 = head.astype(jnp.int32).reshape(NW * NCHUNK, CHUNK)
    tail2 = tail.astype(jnp.int32).reshape(NW * NCHUNK, CHUNK)
    rel2 = relation.astype(jnp.int32).reshape(NW * NCHUNK, CHUNK)
    return _distmult_sc(head2, tail2, rel2, entity_embed, relation_embed)
